# baseline (device time: 694301 ns/iter reference)
import jax
import jax.numpy as jnp
from jax import lax
from jax.experimental import pallas as pl
from jax.experimental.pallas import tpu as pltpu

N_DEV = 8
N_EXPERTS = 32
EPG = N_EXPERTS // N_DEV
CAPACITY = 409


def _ring_neighbors(my):
    left = lax.rem(my + N_DEV - 1, N_DEV)
    right = lax.rem(my + 1, N_DEV)
    return left, right


def _neighbor_barrier(left, right):
    barrier_sem = pltpu.get_barrier_semaphore()
    for nbr in (left, right):
        pl.semaphore_signal(
            barrier_sem, inc=1,
            device_id=(nbr,), device_id_type=pl.DeviceIdType.MESH,
        )
    pl.semaphore_wait(barrier_sem, 2)


def _counts_allgather(counts_pad):

    def body(c_ref, out_ref, comm_ref, send_sems, recv_sems):
        my = lax.axis_index("i")
        left, right = _ring_neighbors(my)
        _neighbor_barrier(left, right)

        out_ref[pl.ds(my, 1), :] = c_ref[:, :]
        comm_ref[0, :, :] = c_ref[:, :]

        for h in range(N_DEV - 1):
            send_slot = h % 2
            recv_slot = (h + 1) % 2
            rdma = pltpu.make_async_remote_copy(
                src_ref=comm_ref.at[send_slot],
                dst_ref=comm_ref.at[recv_slot],
                send_sem=send_sems.at[send_slot],
                recv_sem=recv_sems.at[recv_slot],
                device_id=(right,),
                device_id_type=pl.DeviceIdType.MESH,
            )
            rdma.start()
            rdma.wait()
            origin = lax.rem(my + 2 * N_DEV - h - 1, N_DEV)
            out_ref[pl.ds(origin, 1), :] = comm_ref[recv_slot, :, :]

    return pl.pallas_call(
        body,
        out_shape=jax.ShapeDtypeStruct((N_DEV, 128), jnp.float32),
        in_specs=[pl.BlockSpec(memory_space=pltpu.VMEM)],
        out_specs=pl.BlockSpec(memory_space=pltpu.VMEM),
        scratch_shapes=[
            pltpu.VMEM((2, 1, 128), jnp.float32),
            pltpu.SemaphoreType.DMA((2,)),
            pltpu.SemaphoreType.DMA((2,)),
        ],
        compiler_params=pltpu.CompilerParams(collective_id=0),
    )(counts_pad)


def _moe_ring(x, sel, w_shard):
    n_tok, d_model = x.shape
    d_ff = w_shard.shape[-1]

    def body(x_ref, s_ref, w_ref, out_ref, comm_ref, send_sems, recv_sems):
        my = lax.axis_index("i")
        left, right = _ring_neighbors(my)
        _neighbor_barrier(left, right)

        comm_ref[0] = w_ref[...]
        out_ref[...] = jnp.zeros((n_tok, d_ff), jnp.float32)

        def compute_group(slot, h):
            g = lax.rem(my + 2 * N_DEV - h, N_DEV)
            masks = s_ref[g]
            for j in range(EPG):
                xm = x_ref[...] * masks[:, j : j + 1]
                out_ref[...] += jnp.dot(
                    xm, comm_ref[slot, j], preferred_element_type=jnp.float32
                )

        for h in range(N_DEV - 1):
            send_slot = h % 2
            recv_slot = (h + 1) % 2
            rdma = pltpu.make_async_remote_copy(
                src_ref=comm_ref.at[send_slot],
                dst_ref=comm_ref.at[recv_slot],
                send_sem=send_sems.at[send_slot],
                recv_sem=recv_sems.at[recv_slot],
                device_id=(right,),
                device_id_type=pl.DeviceIdType.MESH,
            )
            rdma.start()
            compute_group(send_slot, h)
            rdma.wait()
        compute_group((N_DEV - 1) % 2, N_DEV - 1)

    return pl.pallas_call(
        body,
        out_shape=jax.ShapeDtypeStruct((n_tok, d_ff), jnp.float32),
        in_specs=[
            pl.BlockSpec(memory_space=pltpu.VMEM),
            pl.BlockSpec(memory_space=pltpu.VMEM),
            pl.BlockSpec(memory_space=pltpu.VMEM),
        ],
        out_specs=pl.BlockSpec(memory_space=pltpu.VMEM),
        scratch_shapes=[
            pltpu.VMEM((2, EPG, d_model, d_ff), jnp.float32),
            pltpu.SemaphoreType.DMA((2,)),
            pltpu.SemaphoreType.DMA((2,)),
        ],
        compiler_params=pltpu.CompilerParams(collective_id=1),
    )(x, sel, w_shard)


def kernel(x, router_W, route_idx, expert_W):
    del router_W
    n_tok = x.shape[0]

    e = route_idx[:, 0]
    oh = (e[:, None] == jnp.arange(N_EXPERTS, dtype=e.dtype)[None, :]).astype(
        jnp.float32
    )

    counts_pad = jnp.zeros((1, 128), jnp.float32).at[0, :N_EXPERTS].set(oh.sum(0))
    all_counts = _counts_allgather(counts_pad)

    my = lax.axis_index("i")
    prev = (jnp.arange(N_DEV) < my).astype(jnp.float32)
    base = (prev @ all_counts)[:N_EXPERTS]
    ranks = jnp.cumsum(oh, axis=0) - oh
    keep = ((base[None, :] + ranks) < CAPACITY) & (oh > 0)
    sel = keep.astype(jnp.float32)

    sel_g = sel.reshape(n_tok, N_DEV, EPG).transpose(1, 0, 2)
    sel_g = jnp.pad(sel_g, ((0, 0), (0, 0), (0, 128 - EPG)))

    return _moe_ring(x, sel_g, expert_W)


# device time: 378315 ns/iter; 1.8352x vs baseline; 1.8352x over previous
import jax
import jax.numpy as jnp
from jax import lax
from jax.experimental import pallas as pl
from jax.experimental.pallas import tpu as pltpu

N_DEV = 8
N_EXPERTS = 32
EPG = N_EXPERTS // N_DEV
CAPACITY = 409


def _ring_neighbors(my):
    left = lax.rem(my + N_DEV - 1, N_DEV)
    right = lax.rem(my + 1, N_DEV)
    return left, right


def _neighbor_barrier(left, right):
    barrier_sem = pltpu.get_barrier_semaphore()
    for nbr in (left, right):
        pl.semaphore_signal(
            barrier_sem, inc=1,
            device_id=(nbr,), device_id_type=pl.DeviceIdType.MESH,
        )
    pl.semaphore_wait(barrier_sem, 2)


def _counts_allgather(counts_pad):

    def body(c_ref, out_ref, comm_ref, send_sems, recv_sems):
        my = lax.axis_index("i")
        left, right = _ring_neighbors(my)
        _neighbor_barrier(left, right)

        out_ref[pl.ds(my, 1), :] = c_ref[:, :]
        comm_ref[0, :, :] = c_ref[:, :]

        for h in range(N_DEV - 1):
            send_slot = h % 2
            recv_slot = (h + 1) % 2
            rdma = pltpu.make_async_remote_copy(
                src_ref=comm_ref.at[send_slot],
                dst_ref=comm_ref.at[recv_slot],
                send_sem=send_sems.at[send_slot],
                recv_sem=recv_sems.at[recv_slot],
                device_id=(right,),
                device_id_type=pl.DeviceIdType.MESH,
            )
            rdma.start()
            rdma.wait()
            origin = lax.rem(my + 2 * N_DEV - h - 1, N_DEV)
            out_ref[pl.ds(origin, 1), :] = comm_ref[recv_slot, :, :]

    return pl.pallas_call(
        body,
        out_shape=jax.ShapeDtypeStruct((N_DEV, 128), jnp.float32),
        in_specs=[pl.BlockSpec(memory_space=pltpu.VMEM)],
        out_specs=pl.BlockSpec(memory_space=pltpu.VMEM),
        scratch_shapes=[
            pltpu.VMEM((2, 1, 128), jnp.float32),
            pltpu.SemaphoreType.DMA((2,)),
            pltpu.SemaphoreType.DMA((2,)),
        ],
        compiler_params=pltpu.CompilerParams(collective_id=0),
    )(counts_pad)


def _moe_ring(x, sel, w_shard):
    n_tok, d_model = x.shape
    d_ff = w_shard.shape[-1]

    def body(x_ref, s_ref, w_ref, out_ref, comm_ref, send_sems, recv_sems):
        my = lax.axis_index("i")
        left, right = _ring_neighbors(my)
        _neighbor_barrier(left, right)

        comm_ref[0] = w_ref[...]
        out_ref[...] = jnp.zeros((n_tok, d_ff), jnp.float32)

        def compute_group(slot, h):
            g = lax.rem(my + 2 * N_DEV - h, N_DEV)
            masks = s_ref[g]
            for j in range(EPG):
                xm = x_ref[...] * masks[:, j : j + 1]
                out_ref[...] += jnp.dot(
                    xm, comm_ref[slot, j], preferred_element_type=jnp.float32
                )

        for h in range(N_DEV - 1):
            send_slot = h % 2
            recv_slot = (h + 1) % 2
            rdma = pltpu.make_async_remote_copy(
                src_ref=comm_ref.at[send_slot],
                dst_ref=comm_ref.at[recv_slot],
                send_sem=send_sems.at[send_slot],
                recv_sem=recv_sems.at[recv_slot],
                device_id=(right,),
                device_id_type=pl.DeviceIdType.MESH,
            )
            rdma.start()
            compute_group(send_slot, h)
            rdma.wait()
        compute_group((N_DEV - 1) % 2, N_DEV - 1)

    return pl.pallas_call(
        body,
        out_shape=jax.ShapeDtypeStruct((n_tok, d_ff), jnp.float32),
        in_specs=[
            pl.BlockSpec(memory_space=pltpu.VMEM),
            pl.BlockSpec(memory_space=pltpu.VMEM),
            pl.BlockSpec(memory_space=pltpu.VMEM),
        ],
        out_specs=pl.BlockSpec(memory_space=pltpu.VMEM),
        scratch_shapes=[
            pltpu.VMEM((2, EPG, d_model, d_ff), jnp.bfloat16),
            pltpu.SemaphoreType.DMA((2,)),
            pltpu.SemaphoreType.DMA((2,)),
        ],
        compiler_params=pltpu.CompilerParams(collective_id=1),
    )(x, sel, w_shard)


def kernel(x, router_W, route_idx, expert_W):
    del router_W
    n_tok = x.shape[0]

    e = route_idx[:, 0]
    oh = (e[:, None] == jnp.arange(N_EXPERTS, dtype=e.dtype)[None, :]).astype(
        jnp.float32
    )

    counts_pad = jnp.zeros((1, 128), jnp.float32).at[0, :N_EXPERTS].set(oh.sum(0))
    all_counts = _counts_allgather(counts_pad)

    my = lax.axis_index("i")
    prev = (jnp.arange(N_DEV) < my).astype(jnp.float32)
    base = (prev @ all_counts)[:N_EXPERTS]
    ranks = jnp.cumsum(oh, axis=0) - oh
    keep = ((base[None, :] + ranks) < CAPACITY) & (oh > 0)
    sel = keep.astype(jnp.float32)

    sel_g = sel.reshape(n_tok, N_DEV, EPG).transpose(1, 0, 2)
    sel_g = jnp.pad(sel_g, ((0, 0), (0, 0), (0, 128 - EPG)))

    return _moe_ring(
        x.astype(jnp.bfloat16),
        sel_g.astype(jnp.bfloat16),
        expert_W.astype(jnp.bfloat16),
    )


# device time: 223773 ns/iter; 3.1027x vs baseline; 1.6906x over previous
import jax
import jax.numpy as jnp
from jax import lax
from jax.experimental import pallas as pl
from jax.experimental.pallas import tpu as pltpu

N_DEV = 8
N_EXPERTS = 32
EPG = N_EXPERTS // N_DEV
CAPACITY = 409


def _ring_neighbors(my):
    left = lax.rem(my + N_DEV - 1, N_DEV)
    right = lax.rem(my + 1, N_DEV)
    return left, right


def _neighbor_barrier(left, right):
    barrier_sem = pltpu.get_barrier_semaphore()
    for nbr in (left, right):
        pl.semaphore_signal(
            barrier_sem, inc=1,
            device_id=(nbr,), device_id_type=pl.DeviceIdType.MESH,
        )
    pl.semaphore_wait(barrier_sem, 2)


def _counts_allgather(counts_pad):

    def body(c_ref, out_ref, comm_ref, send_sems, recv_sems):
        my = lax.axis_index("i")
        left, right = _ring_neighbors(my)
        _neighbor_barrier(left, right)

        out_ref[pl.ds(my, 1), :] = c_ref[:, :]
        comm_ref[0, :, :] = c_ref[:, :]

        for h in range(N_DEV - 1):
            send_slot = h % 2
            recv_slot = (h + 1) % 2
            rdma = pltpu.make_async_remote_copy(
                src_ref=comm_ref.at[send_slot],
                dst_ref=comm_ref.at[recv_slot],
                send_sem=send_sems.at[send_slot],
                recv_sem=recv_sems.at[recv_slot],
                device_id=(right,),
                device_id_type=pl.DeviceIdType.MESH,
            )
            rdma.start()
            rdma.wait()
            origin = lax.rem(my + 2 * N_DEV - h - 1, N_DEV)
            out_ref[pl.ds(origin, 1), :] = comm_ref[recv_slot, :, :]

    return pl.pallas_call(
        body,
        out_shape=jax.ShapeDtypeStruct((N_DEV, 128), jnp.float32),
        in_specs=[pl.BlockSpec(memory_space=pltpu.VMEM)],
        out_specs=pl.BlockSpec(memory_space=pltpu.VMEM),
        scratch_shapes=[
            pltpu.VMEM((2, 1, 128), jnp.float32),
            pltpu.SemaphoreType.DMA((2,)),
            pltpu.SemaphoreType.DMA((2,)),
        ],
        compiler_params=pltpu.CompilerParams(collective_id=0),
    )(counts_pad)


def _moe_ring(x, sel, w_shard):
    n_tok, d_model = x.shape
    d_ff = w_shard.shape[-1]

    def body(
        x_ref, s_ref, w_ref, out_ref,
        comm_r, comm_l, ss_r, rs_r, ss_l, rs_l,
    ):
        my = lax.axis_index("i")
        left, right = _ring_neighbors(my)
        _neighbor_barrier(left, right)

        comm_r[0] = w_ref[0:2]
        comm_l[0] = w_ref[2:4]
        out_ref[...] = jnp.zeros((n_tok, d_ff), jnp.float32)

        def compute_groups(slot, h):
            g_r = lax.rem(my + 2 * N_DEV - h, N_DEV)
            g_l = lax.rem(my + h, N_DEV)
            m_r = s_ref[g_r]
            m_l = s_ref[g_l]
            for j in range(2):
                out_ref[...] += jnp.dot(
                    x_ref[...] * m_r[:, j : j + 1],
                    comm_r[slot, j],
                    preferred_element_type=jnp.float32,
                )
                out_ref[...] += jnp.dot(
                    x_ref[...] * m_l[:, 2 + j : 3 + j],
                    comm_l[slot, j],
                    preferred_element_type=jnp.float32,
                )

        for h in range(N_DEV - 1):
            ss = h % 2
            rs = (h + 1) % 2
            rdma_r = pltpu.make_async_remote_copy(
                src_ref=comm_r.at[ss],
                dst_ref=comm_r.at[rs],
                send_sem=ss_r.at[ss],
                recv_sem=rs_r.at[rs],
                device_id=(right,),
                device_id_type=pl.DeviceIdType.MESH,
            )
            rdma_l = pltpu.make_async_remote_copy(
                src_ref=comm_l.at[ss],
                dst_ref=comm_l.at[rs],
                send_sem=ss_l.at[ss],
                recv_sem=rs_l.at[rs],
                device_id=(left,),
                device_id_type=pl.DeviceIdType.MESH,
            )
            rdma_r.start()
            rdma_l.start()
            compute_groups(ss, h)
            rdma_r.wait()
            rdma_l.wait()
        compute_groups((N_DEV - 1) % 2, N_DEV - 1)

    return pl.pallas_call(
        body,
        out_shape=jax.ShapeDtypeStruct((n_tok, d_ff), jnp.float32),
        in_specs=[
            pl.BlockSpec(memory_space=pltpu.VMEM),
            pl.BlockSpec(memory_space=pltpu.VMEM),
            pl.BlockSpec(memory_space=pltpu.VMEM),
        ],
        out_specs=pl.BlockSpec(memory_space=pltpu.VMEM),
        scratch_shapes=[
            pltpu.VMEM((2, EPG // 2, d_model, d_ff), jnp.bfloat16),
            pltpu.VMEM((2, EPG // 2, d_model, d_ff), jnp.bfloat16),
            pltpu.SemaphoreType.DMA((2,)),
            pltpu.SemaphoreType.DMA((2,)),
            pltpu.SemaphoreType.DMA((2,)),
            pltpu.SemaphoreType.DMA((2,)),
        ],
        compiler_params=pltpu.CompilerParams(collective_id=1),
    )(x, sel, w_shard)


def kernel(x, router_W, route_idx, expert_W):
    del router_W
    n_tok = x.shape[0]

    e = route_idx[:, 0]
    oh = (e[:, None] == jnp.arange(N_EXPERTS, dtype=e.dtype)[None, :]).astype(
        jnp.float32
    )

    counts_pad = jnp.zeros((1, 128), jnp.float32).at[0, :N_EXPERTS].set(oh.sum(0))
    all_counts = _counts_allgather(counts_pad)

    my = lax.axis_index("i")
    prev = (jnp.arange(N_DEV) < my).astype(jnp.float32)
    base = (prev @ all_counts)[:N_EXPERTS]
    ranks = jnp.cumsum(oh, axis=0) - oh
    keep = ((base[None, :] + ranks) < CAPACITY) & (oh > 0)
    sel = keep.astype(jnp.float32)

    sel_g = sel.reshape(n_tok, N_DEV, EPG).transpose(1, 0, 2)
    sel_g = jnp.pad(sel_g, ((0, 0), (0, 0), (0, 128 - EPG)))

    return _moe_ring(
        x.astype(jnp.bfloat16),
        sel_g.astype(jnp.bfloat16),
        expert_W.astype(jnp.bfloat16),
    )
